# baseline (speedup 1.0000x reference)
import functools

import jax
import jax.numpy as jnp
from jax import lax
from jax.experimental import pallas as pl
from jax.experimental.pallas import tpu as pltpu
from jax.experimental.pallas import tpu_sc as plsc

N_NODES = 10000
N_EDGES = 320000
LANES = 16
NC = 2
NS = 16
NW = NC * NS
K_EDGES = 128


N_EDGES_PAD = 327680


def _sc_spmm(h, packed, d_feat, k_edges, NBUF):
    npt = N_EDGES_PAD // (NW * k_edges)
    assert npt % NBUF == 0 and NBUF in (4, 5)
    row_stride = 624
    row_width = N_NODES - row_stride * (NS - 1)
    mesh = plsc.VectorSubcoreMesh(core_axis_name="c", subcore_axis_name="s",
                                  num_cores=NC, num_subcores=NS)

    @functools.partial(
        pl.kernel,
        out_type=jax.ShapeDtypeStruct((NC, N_NODES, d_feat), jnp.float32),
        mesh=mesh,
        scratch_types=[
            [pltpu.VMEM((3, k_edges), jnp.int32) for _ in range(NBUF)],
            [pltpu.VMEM((k_edges, d_feat), jnp.float32) for _ in range(NBUF)],
            pltpu.VMEM_SHARED((N_NODES, d_feat), jnp.float32),
            [pltpu.SemaphoreType.DMA for _ in range(NBUF)],
            [pltpu.SemaphoreType.DMA for _ in range(NBUF)],
            [pltpu.SemaphoreType.DMA for _ in range(NBUF)],
        ],
        compiler_params=pltpu.CompilerParams(use_tc_tiling_on_sc=False, needs_layout_passes=False),
    )
    def spmm(h_hbm, packed_hbm, zero_hbm, out_hbm,
             ebuf, rows, acc_sh, isem, gsem, ssem):
        cid = lax.axis_index("c")
        sid = lax.axis_index("s")
        wid = sid * NC + cid
        chunk0 = wid * npt

        row0 = sid * row_stride
        pltpu.sync_copy(zero_hbm.at[pl.ds(row0, row_width)],
                        acc_sh.at[pl.ds(row0, row_width)])
        plsc.subcore_barrier()

        def idx_load(i, s):
            pltpu.async_copy(packed_hbm.at[chunk0 + i], ebuf[s], isem[s])

        def wait_idx(s):
            pltpu.make_async_copy(packed_hbm.at[chunk0], ebuf[s], isem[s]).wait()

        def gather(i, s):
            pltpu.async_copy(h_hbm.at[ebuf[s].at[0]], rows[s], gsem[s])

        def wait_gather(s):
            pltpu.make_async_copy(h_hbm.at[ebuf[s].at[0]], rows[s],
                                  gsem[s]).wait()

        def scatter(s):
            pltpu.async_copy(rows[s], acc_sh.at[ebuf[s].at[1]], ssem[s],
                             add=True)

        def wait_scatter(s):
            pltpu.make_async_copy(rows[s], acc_sh.at[ebuf[s].at[1]],
                                  ssem[s]).wait()

        def scale(s):
            def scale_group(g2, carry2):
                wbits = ebuf[s][2, pl.ds(g2 * LANES, LANES)]
                w16 = plsc.bitcast(wbits, jnp.float32)
                for e in range(LANES):
                    idx = jnp.full((LANES,), e, dtype=jnp.int32)
                    wb = jnp.take(w16, idx)
                    row = g2 * LANES + e
                    for j in range(d_feat // LANES):
                        sl = pl.ds(j * LANES, LANES)
                        rows[s][row, sl] = rows[s][row, sl] * wb
                return carry2

            lax.fori_loop(0, k_edges // LANES, scale_group, 0, unroll=False)

        idx_load(0, 0)
        idx_load(1, 1)
        idx_load(2, 2)
        wait_idx(0)
        gather(0, 0)
        wait_idx(1)
        gather(1, 1)

        def outer(g, carry):
            for b in range(NBUF):
                i = g * NBUF + b
                s3 = (b + 3) % NBUF
                s2 = (b + 2) % NBUF

                @pl.when(i + 3 < npt)
                def _():
                    @pl.when(i >= NBUF - 3)
                    def _():
                        wait_scatter(s3)
                    idx_load(i + 3, s3)

                @pl.when(i + 2 < npt)
                def _():
                    wait_idx(s2)
                    gather(i + 2, s2)

                wait_gather(b)
                scale(b)
                scatter(b)
            return carry

        lax.fori_loop(0, npt // NBUF, outer, 0, unroll=False)

        for s in range(NBUF):
            wait_scatter(s)

        plsc.subcore_barrier()
        pltpu.sync_copy(acc_sh.at[pl.ds(row0, row_width)],
                        out_hbm.at[cid].at[pl.ds(row0, row_width)])

    zeros = jnp.zeros((N_NODES, d_feat), jnp.float32)
    return spmm(h, packed, zeros)


def _tc_matmul(x, w):
    n, d_in = x.shape
    d_out = w.shape[1]
    br = 2000

    def mm(x_ref, w_ref, o_ref):
        o_ref[...] = jnp.dot(x_ref[...], w_ref[...],
                             preferred_element_type=jnp.float32)

    return pl.pallas_call(
        mm,
        out_shape=jax.ShapeDtypeStruct((n, d_out), jnp.float32),
        grid=(n // br,),
        in_specs=[
            pl.BlockSpec((br, d_in), lambda i: (i, 0)),
            pl.BlockSpec((d_in, d_out), lambda i: (0, 0)),
        ],
        out_specs=pl.BlockSpec((br, d_out), lambda i: (i, 0)),
    )(x, w)


def _tc_combine_relu_matmul(parts, b, w):
    _, n, d_in = parts.shape
    d_out = w.shape[1]
    br = 2000

    def body(p_ref, b_ref, w_ref, o_ref):
        z = jax.nn.relu(p_ref[0] + p_ref[1] + b_ref[...])
        o_ref[...] = jnp.dot(z, w_ref[...], preferred_element_type=jnp.float32)

    return pl.pallas_call(
        body,
        out_shape=jax.ShapeDtypeStruct((n, d_out), jnp.float32),
        grid=(n // br,),
        in_specs=[
            pl.BlockSpec((2, br, d_in), lambda i: (0, i, 0)),
            pl.BlockSpec((1, d_in), lambda i: (0, 0)),
            pl.BlockSpec((d_in, d_out), lambda i: (0, 0)),
        ],
        out_specs=pl.BlockSpec((br, d_out), lambda i: (i, 0)),
    )(parts, b.reshape(1, -1), w)


def _tc_combine_bias(parts, b):
    _, n, d = parts.shape
    br = 2000

    def body(p_ref, b_ref, o_ref):
        o_ref[...] = p_ref[0] + p_ref[1] + b_ref[...]

    return pl.pallas_call(
        body,
        out_shape=jax.ShapeDtypeStruct((n, d), jnp.float32),
        grid=(n // br,),
        in_specs=[
            pl.BlockSpec((2, br, d), lambda i: (0, i, 0)),
            pl.BlockSpec((1, d), lambda i: (0, 0)),
        ],
        out_specs=pl.BlockSpec((br, d), lambda i: (i, 0)),
    )(parts, b.reshape(1, -1))


def kernel(features, edge_index, edge_weight, W1, b1, W2, b2):
    pad = N_EDGES_PAD - N_EDGES
    src = jnp.pad(edge_index[0].astype(jnp.int32), (0, pad))
    dst = jnp.pad(edge_index[1].astype(jnp.int32), (0, pad))
    wbits = jax.lax.bitcast_convert_type(
        jnp.pad(edge_weight, (0, pad)), jnp.int32)

    def pack(k):
        return jnp.stack([src.reshape(-1, k), dst.reshape(-1, k),
                          wbits.reshape(-1, k)], axis=1)

    h1 = _tc_matmul(features, W1)
    p1 = _sc_spmm(h1, pack(80), W1.shape[1], 80, NBUF=4)
    h2 = _tc_combine_relu_matmul(p1, b1, W2)
    p2 = _sc_spmm(h2, pack(128), W2.shape[1], 128, NBUF=5)
    return _tc_combine_bias(p2, b2)


# speedup vs baseline: 1.0624x; 1.0624x over previous
import functools

import jax
import jax.numpy as jnp
from jax import lax
from jax.experimental import pallas as pl
from jax.experimental.pallas import tpu as pltpu
from jax.experimental.pallas import tpu_sc as plsc

N_NODES = 10000
N_EDGES = 320000
LANES = 16
NC = 2
NS = 16
NW = NC * NS
K_EDGES = 128


N_EDGES_PAD = 327680


def _sc_spmm(h, packed, d_feat, k_edges):
    npt = N_EDGES_PAD // (NW * k_edges)
    assert npt % 2 == 0
    row_stride = 624
    row_width = N_NODES - row_stride * (NS - 1)
    mesh = plsc.VectorSubcoreMesh(core_axis_name="c", subcore_axis_name="s",
                                  num_cores=NC, num_subcores=NS)

    @functools.partial(
        pl.kernel,
        out_type=jax.ShapeDtypeStruct((NC, N_NODES, d_feat), jnp.float32),
        mesh=mesh,
        scratch_types=[
            [pltpu.VMEM((3, k_edges), jnp.int32) for _ in range(2)],
            [pltpu.VMEM((k_edges, d_feat), jnp.float32) for _ in range(2)],
            pltpu.VMEM_SHARED((N_NODES, d_feat), jnp.float32),
            [pltpu.SemaphoreType.DMA for _ in range(2)],
            [pltpu.SemaphoreType.DMA for _ in range(2)],
        ],
        compiler_params=pltpu.CompilerParams(use_tc_tiling_on_sc=False, needs_layout_passes=False),
    )
    def spmm(h_hbm, packed_hbm, zero_hbm, out_hbm,
             ebuf, rows, acc_sh, isem, gsem):
        cid = lax.axis_index("c")
        sid = lax.axis_index("s")
        wid = sid * NC + cid
        chunk0 = wid * npt

        row0 = sid * row_stride
        pltpu.sync_copy(zero_hbm.at[pl.ds(row0, row_width)],
                        acc_sh.at[pl.ds(row0, row_width)])
        plsc.subcore_barrier()

        def idx_load(i, s):
            pltpu.async_copy(packed_hbm.at[chunk0 + i], ebuf[s], isem[s])

        def wait_idx(s):
            pltpu.make_async_copy(packed_hbm.at[chunk0], ebuf[s], isem[s]).wait()

        def gather(i, s):
            pltpu.async_copy(h_hbm.at[ebuf[s].at[0]], rows[s], gsem[s])

        def wait_gather(s):
            pltpu.make_async_copy(h_hbm.at[ebuf[s].at[0]], rows[s],
                                  gsem[s]).wait()

        def scale(s):
            def scale_group(g2, carry2):
                wbits = ebuf[s][2, pl.ds(g2 * LANES, LANES)]
                w16 = plsc.bitcast(wbits, jnp.float32)
                for e in range(LANES):
                    idx = jnp.full((LANES,), e, dtype=jnp.int32)
                    wb = jnp.take(w16, idx)
                    row = g2 * LANES + e
                    for j in range(d_feat // LANES):
                        sl = pl.ds(j * LANES, LANES)
                        rows[s][row, sl] = rows[s][row, sl] * wb
                return carry2

            lax.fori_loop(0, k_edges // LANES, scale_group, 0, unroll=False)

        idx_load(0, 0)
        idx_load(1, 1)
        wait_idx(0)
        gather(0, 0)

        def outer(g, carry):
            for b in range(2):
                i = g * 2 + b
                b1 = (b + 1) % 2
                @pl.when(i + 1 < npt)
                def _():
                    wait_idx(b1)
                    gather(i + 1, b1)

                wait_gather(b)
                scale(b)
                pltpu.sync_copy(rows[b], acc_sh.at[ebuf[b].at[1]], add=True)
                @pl.when(i + 2 < npt)
                def _():
                    idx_load(i + 2, b)
            return carry

        lax.fori_loop(0, npt // 2, outer, 0, unroll=False)

        plsc.subcore_barrier()
        pltpu.sync_copy(acc_sh.at[pl.ds(row0, row_width)],
                        out_hbm.at[cid].at[pl.ds(row0, row_width)])

    zeros = jnp.zeros((N_NODES, d_feat), jnp.float32)
    return spmm(h, packed, zeros)


def _tc_matmul(x, w):
    n, d_in = x.shape
    d_out = w.shape[1]
    br = 2000

    def mm(x_ref, w_ref, o_ref):
        o_ref[...] = jnp.dot(x_ref[...], w_ref[...],
                             preferred_element_type=jnp.float32)

    return pl.pallas_call(
        mm,
        out_shape=jax.ShapeDtypeStruct((n, d_out), jnp.float32),
        grid=(n // br,),
        in_specs=[
            pl.BlockSpec((br, d_in), lambda i: (i, 0)),
            pl.BlockSpec((d_in, d_out), lambda i: (0, 0)),
        ],
        out_specs=pl.BlockSpec((br, d_out), lambda i: (i, 0)),
    )(x, w)


def _tc_combine_relu_matmul(parts, b, w):
    _, n, d_in = parts.shape
    d_out = w.shape[1]
    br = 2000

    def body(p_ref, b_ref, w_ref, o_ref):
        z = jax.nn.relu(p_ref[0] + p_ref[1] + b_ref[...])
        o_ref[...] = jnp.dot(z, w_ref[...], preferred_element_type=jnp.float32)

    return pl.pallas_call(
        body,
        out_shape=jax.ShapeDtypeStruct((n, d_out), jnp.float32),
        grid=(n // br,),
        in_specs=[
            pl.BlockSpec((2, br, d_in), lambda i: (0, i, 0)),
            pl.BlockSpec((1, d_in), lambda i: (0, 0)),
            pl.BlockSpec((d_in, d_out), lambda i: (0, 0)),
        ],
        out_specs=pl.BlockSpec((br, d_out), lambda i: (i, 0)),
    )(parts, b.reshape(1, -1), w)


def _tc_combine_bias(parts, b):
    _, n, d = parts.shape
    br = 2000

    def body(p_ref, b_ref, o_ref):
        o_ref[...] = p_ref[0] + p_ref[1] + b_ref[...]

    return pl.pallas_call(
        body,
        out_shape=jax.ShapeDtypeStruct((n, d), jnp.float32),
        grid=(n // br,),
        in_specs=[
            pl.BlockSpec((2, br, d), lambda i: (0, i, 0)),
            pl.BlockSpec((1, d), lambda i: (0, 0)),
        ],
        out_specs=pl.BlockSpec((br, d), lambda i: (i, 0)),
    )(parts, b.reshape(1, -1))


def kernel(features, edge_index, edge_weight, W1, b1, W2, b2):
    pad = N_EDGES_PAD - N_EDGES
    src = jnp.pad(edge_index[0].astype(jnp.int32), (0, pad))
    dst = jnp.pad(edge_index[1].astype(jnp.int32), (0, pad))
    wbits = jax.lax.bitcast_convert_type(
        jnp.pad(edge_weight, (0, pad)), jnp.int32)

    def pack(k):
        return jnp.stack([src.reshape(-1, k), dst.reshape(-1, k),
                          wbits.reshape(-1, k)], axis=1)

    h1 = _tc_matmul(features, W1)
    p1 = _sc_spmm(h1, pack(128), W1.shape[1], 128)
    h2 = _tc_combine_relu_matmul(p1, b1, W2)
    p2 = _sc_spmm(h2, pack(128), W2.shape[1], 128)
    return _tc_combine_bias(p2, b2)
